# baseline (speedup 1.0000x reference)
import functools

import jax
import jax.numpy as jnp
from jax import lax
from jax.experimental import pallas as pl
from jax.experimental.pallas import tpu as pltpu
from jax.experimental.pallas import tpu_sc as plsc

_COST_POINT = 0.1
_QT = 512
_NQ = 4096
_NG = 1024
_CR = 32


def _cost_body(coords_ref, probs_ref, gtc_ref, lab_ref, maskf_ref, c_ref):
    px = coords_ref[0, :, 0:1]
    py = coords_ref[0, :, 1:2]
    gx = gtc_ref[0, 0:1, :]
    gy = gtc_ref[0, 1:2, :]
    dx = px - gx
    dy = py - gy
    dist = jnp.sqrt(dx * dx + dy * dy)

    probs = probs_ref[0, :, :]
    lab = lab_ref[0, :, :]
    b0 = (lab & 1) != 0
    b1 = (lab & 2) != 0
    b2 = lab >= 4
    p = [probs[:, c:c + 1] for c in range(6)]
    q01 = jnp.where(b0, p[1], p[0])
    q23 = jnp.where(b0, p[3], p[2])
    q45 = jnp.where(b0, p[5], p[4])
    gathered = jnp.where(b2, q45, jnp.where(b1, q23, q01))

    C = _COST_POINT * dist + (-gathered)
    c_ref[0, :, :] = jnp.where(maskf_ref[0, :, :] > 0, C, jnp.inf)


def _insert8(t, ix, v, qi):
    b0 = v < t[0]
    b1 = v < t[1]
    b2 = v < t[2]
    b3 = v < t[3]
    t3n = jnp.where(b3, jnp.where(b2, t[2], v), t[3])
    i3n = jnp.where(b3, jnp.where(b2, ix[2], qi), ix[3])
    t2n = jnp.where(b2, jnp.where(b1, t[1], v), t[2])
    i2n = jnp.where(b2, jnp.where(b1, ix[1], qi), ix[2])
    t1n = jnp.where(b1, jnp.where(b0, t[0], v), t[1])
    i1n = jnp.where(b1, jnp.where(b0, ix[0], qi), ix[1])
    t0n = jnp.where(b0, v, t[0])
    i0n = jnp.where(b0, qi, ix[0])
    return [t0n, t1n, t2n, t3n], [i0n, i1n, i2n, i3n]


def _sc_body(chbm, out_hbm, tv, ti, buf0, buf1, shv, shi, mbv, mbi,
             sem0, sem1):
    c = lax.axis_index("c")
    s = lax.axis_index("s")
    b = c * 4 + s // 4
    qslot = s % 4
    qbase = qslot * 1024
    zeros16 = jnp.zeros((16,), jnp.int32)
    inf16 = jnp.full((16,), jnp.inf, jnp.float32)

    def init_g(g, _):
        sl = pl.ds(g * 16, 16)
        for j in range(4):
            tv[j, sl] = inf16
            ti[j, sl] = zeros16 + (qbase + j)
        return 0
    lax.fori_loop(0, _NG // 16, init_g, 0)

    nch = 1024 // _CR
    pltpu.async_copy(chbm.at[b, pl.ds(qbase, _CR), :], buf0, sem0)
    pltpu.async_copy(chbm.at[b, pl.ds(qbase + _CR, _CR), :], buf1, sem1)

    def scan_chunk(buf, k):
        def g_body(g, _):
            sl = pl.ds(g * 16, 16)
            for r8 in range(_CR // 8):
                t3 = tv[3, sl]
                m = buf[r8 * 8, sl]
                for i in range(1, 8):
                    m = jnp.minimum(m, buf[r8 * 8 + i, sl])
                cnt = plsc.all_reduce_population_count(m < t3)
                anyhit = cnt[0]

                @pl.when(anyhit > 0)
                def _insert():
                    t = [tv[j, sl] for j in range(4)]
                    ix = [ti[j, sl] for j in range(4)]
                    for i in range(8):
                        v = buf[r8 * 8 + i, sl]
                        qi = zeros16 + (qbase + k * _CR + r8 * 8 + i)
                        t, ix = _insert8(t, ix, v, qi)
                    for j in range(4):
                        tv[j, sl] = t[j]
                        ti[j, sl] = ix[j]
            return 0
        lax.fori_loop(0, _NG // 16, g_body, 0)

    def outer(k2, _):
        for (buf, sem, off) in ((buf0, sem0, 0), (buf1, sem1, 1)):
            k = k2 * 2 + off
            pltpu.make_async_copy(
                chbm.at[b, pl.ds(qbase + k * _CR, _CR), :], buf, sem).wait()
            scan_chunk(buf, k)

            @pl.when(k + 2 < nch)
            def _prefetch():
                pltpu.async_copy(
                    chbm.at[b, pl.ds(qbase + (k + 2) * _CR, _CR), :], buf, sem)
        return 0
    lax.fori_loop(0, nch // 2, outer, 0)

    pltpu.sync_copy(tv, shv.at[s])
    pltpu.sync_copy(ti, shi.at[s])
    plsc.subcore_barrier()

    @pl.when(qslot == 0)
    def _merge():
        for w in range(1, 4):
            pltpu.sync_copy(shv.at[s + w], mbv)
            pltpu.sync_copy(shi.at[s + w], mbi)

            def mg(g, _):
                sl = pl.ds(g * 16, 16)
                t = [tv[j, sl] for j in range(4)]
                ix = [ti[j, sl] for j in range(4)]
                for j in range(4):
                    t, ix = _insert8(t, ix, mbv[j, sl], mbi[j, sl])
                for j in range(4):
                    tv[j, sl] = t[j]
                    ti[j, sl] = ix[j]
                return 0
            lax.fori_loop(0, _NG // 16, mg, 0)
        pltpu.sync_copy(ti, out_hbm.at[b])


@jax.jit
def kernel(pred_coords, pred_logits, gt_coords, gt_labels, gt_masks):
    bs, nq, _ = pred_coords.shape
    ng = gt_coords.shape[1]
    pred_probs = jax.nn.softmax(pred_logits, axis=-1)
    gtc_t = jnp.swapaxes(gt_coords, 1, 2)
    lab = gt_labels.astype(jnp.int32).reshape(bs, 1, ng)
    maskf = gt_masks.astype(jnp.float32).reshape(bs, 1, ng)

    cmat = pl.pallas_call(
        _cost_body,
        grid=(bs, nq // _QT),
        in_specs=[
            pl.BlockSpec((1, _QT, 2), lambda b, q: (b, q, 0)),
            pl.BlockSpec((1, _QT, 6), lambda b, q: (b, q, 0)),
            pl.BlockSpec((1, 2, ng), lambda b, q: (b, 0, 0)),
            pl.BlockSpec((1, 1, ng), lambda b, q: (b, 0, 0)),
            pl.BlockSpec((1, 1, ng), lambda b, q: (b, 0, 0)),
        ],
        out_specs=pl.BlockSpec((1, _QT, ng), lambda b, q: (b, q, 0)),
        out_shape=jax.ShapeDtypeStruct((bs, nq, ng), jnp.float32),
        compiler_params=pltpu.CompilerParams(
            dimension_semantics=("parallel", "arbitrary")),
    )(pred_coords, pred_probs, gtc_t, lab, maskf)

    sc_fn = pl.kernel(
        _sc_body,
        out_type=jax.ShapeDtypeStruct((bs, 4, ng), jnp.int32),
        mesh=plsc.VectorSubcoreMesh(core_axis_name="c", subcore_axis_name="s",
                                    num_cores=2, num_subcores=16),
        compiler_params=pltpu.CompilerParams(needs_layout_passes=False),
        scratch_types=[
            pltpu.VMEM((4, ng), jnp.float32),
            pltpu.VMEM((4, ng), jnp.int32),
            pltpu.VMEM((_CR, ng), jnp.float32),
            pltpu.VMEM((_CR, ng), jnp.float32),
            pltpu.VMEM_SHARED((16, 4, ng), jnp.float32),
            pltpu.VMEM_SHARED((16, 4, ng), jnp.int32),
            pltpu.VMEM((4, ng), jnp.float32),
            pltpu.VMEM((4, ng), jnp.int32),
            pltpu.SemaphoreType.DMA,
            pltpu.SemaphoreType.DMA,
        ],
    )
    return sc_fn(cmat)
